# Initial kernel scaffold; baseline (speedup 1.0000x reference)
import jax
import jax.numpy as jnp
from jax.experimental import pallas as pl


def kernel(x, edge_index, adj_values):
    raise NotImplementedError("write your pallas kernel here")


# speedup vs baseline: 4.6507x; 4.6507x over previous
import functools

import jax
import jax.numpy as jnp
from jax import lax
from jax.experimental import pallas as pl
from jax.experimental.pallas import tpu as pltpu
from jax.experimental.pallas import tpu_sc as plsc

N = 10000
D = 128
MIN_NORM = 1e-15
BALL_EPS = 4e-3

NC = 2
NS = 16
NW = NC * NS
K = 128
N_PAD = 10240
ROWS_PER_TILE = N_PAD // NS
LANES = 16


def _logmap0_body(x_ref, o_ref):
    x = x_ref[...]
    norm = jnp.sqrt(jnp.sum(x * x, axis=-1, keepdims=True))
    norm = jnp.maximum(norm, MIN_NORM)
    z = jnp.clip(norm, -1.0 + 1e-7, 1.0 - 1e-7)
    artanh = 0.5 * (jnp.log(1.0 + z) - jnp.log(1.0 - z))
    o_ref[...] = (artanh / norm) * x


def _logmap0_tc(x):
    return pl.pallas_call(
        _logmap0_body,
        out_shape=jax.ShapeDtypeStruct(x.shape, x.dtype),
    )(x)


def _expmap_proj_body(p_ref, o_ref):
    u = p_ref[0] + p_ref[1]
    un = jnp.maximum(jnp.sqrt(jnp.sum(u * u, axis=-1, keepdims=True)), MIN_NORM)
    y = jnp.tanh(un) * u / un
    yn = jnp.maximum(jnp.sqrt(jnp.sum(y * y, axis=-1, keepdims=True)), MIN_NORM)
    maxnorm = 1.0 - BALL_EPS
    o_ref[...] = jnp.where(yn > maxnorm, y / yn * maxnorm, y)


def _expmap_proj_tc(parts):
    return pl.pallas_call(
        _expmap_proj_body,
        out_shape=jax.ShapeDtypeStruct((N, D), jnp.float32),
    )(parts)


def _spmm_sc(xt, src3, dst3, adj3, chunks):
    mesh = plsc.VectorSubcoreMesh(
        core_axis_name="c", subcore_axis_name="s",
        num_cores=NC, num_subcores=NS,
    )

    @functools.partial(
        pl.kernel,
        out_type=jax.ShapeDtypeStruct((NC, N_PAD, D), jnp.float32),
        mesh=mesh,
        scratch_types=[
            pltpu.VMEM((chunks, K), jnp.int32),
            pltpu.VMEM((chunks, K), jnp.int32),
            pltpu.VMEM((chunks, K), jnp.float32),
            pltpu.VMEM((K, D), jnp.float32),
            pltpu.VMEM_SHARED((N_PAD, D), jnp.float32),
            pltpu.SemaphoreType.DMA,
        ],
    )
    def spmm(xt_hbm, src_hbm, dst_hbm, adj_hbm, out_hbm,
             src_v, dst_v, adj_v, rows_v, acc, sem):
        c = lax.axis_index("c")
        s = lax.axis_index("s")
        wid = c * NS + s

        def _zero_row(r, carry):
            for f in range(D // LANES):
                rows_v[r, pl.ds(f * LANES, LANES)] = jnp.zeros((LANES,), jnp.float32)
            return carry
        lax.fori_loop(0, K, _zero_row, 0)
        base = s * ROWS_PER_TILE
        for i in range(ROWS_PER_TILE // K):
            pltpu.sync_copy(rows_v, acc.at[pl.ds(base + i * K, K)])

        pltpu.sync_copy(src_hbm.at[wid], src_v)
        pltpu.sync_copy(dst_hbm.at[wid], dst_v)
        pltpu.sync_copy(adj_hbm.at[wid], adj_v)
        plsc.subcore_barrier()

        def _chunk(j, carry):
            pltpu.async_copy(xt_hbm.at[src_v.at[j]], rows_v, sem).wait()

            def _scale(g, inner):
                a16 = adj_v[j, pl.ds(g * LANES, LANES)]
                for l in range(LANES):
                    a = jnp.full((LANES,), a16[l])
                    e = g * LANES + l
                    for f in range(D // LANES):
                        sl = pl.ds(f * LANES, LANES)
                        rows_v[e, sl] = rows_v[e, sl] * a
                return inner
            lax.fori_loop(0, K // LANES, _scale, 0)

            pltpu.sync_copy(rows_v, acc.at[dst_v.at[j]], add=True)
            return carry
        lax.fori_loop(0, chunks, _chunk, 0)

        plsc.subcore_barrier()
        pltpu.sync_copy(acc.at[pl.ds(base, ROWS_PER_TILE)],
                        out_hbm.at[c, pl.ds(base, ROWS_PER_TILE)])

    return spmm(xt, src3, dst3, adj3)


def kernel(x, edge_index, adj_values):
    E = edge_index.shape[1]
    epw = -(-E // (NW * K)) * K
    e_pad = epw * NW
    chunks = epw // K

    src = edge_index[0].astype(jnp.int32)
    dst = edge_index[1].astype(jnp.int32)
    adj = adj_values.astype(jnp.float32)
    pad = e_pad - E
    if pad:
        src = jnp.concatenate([src, jnp.zeros((pad,), jnp.int32)])
        dst = jnp.concatenate([dst, jnp.zeros((pad,), jnp.int32)])
        adj = jnp.concatenate([adj, jnp.zeros((pad,), jnp.float32)])
    src3 = src.reshape(NW, chunks, K)
    dst3 = dst.reshape(NW, chunks, K)
    adj3 = adj.reshape(NW, chunks, K)

    xt = _logmap0_tc(x)
    parts = _spmm_sc(xt, src3, dst3, adj3, chunks)
    return _expmap_proj_tc(parts[:, :N, :])
